# baseline (speedup 1.0000x reference)
import functools

import jax
import jax.numpy as jnp
from jax import lax
from jax.experimental import pallas as pl
from jax.experimental.pallas import tpu as pltpu
from jax.experimental.pallas import tpu_sc as plsc

_CB = 8192
_D = 32
_TM = 256
_KC = 256
_NKC = _CB // _KC
_RND = 4096 // _KC


def _c2_body(ct_ref, c2_ref):
    c = ct_ref[...]
    c2_ref[...] = jnp.sum(c * c, axis=0, keepdims=True)


def _sq_norms(ct):
    return pl.pallas_call(
        _c2_body,
        grid=(1,),
        in_specs=[pl.BlockSpec((_D, _CB), lambda i: (0, 0))],
        out_specs=pl.BlockSpec((1, _CB), lambda i: (0, 0)),
        out_shape=jax.ShapeDtypeStruct((1, _CB), jnp.float32),
    )(ct)


def _vq_argmin_body(zt_ref, ctm2_ref, c2_ref, tok_ref):
    zt = zt_ref[...]
    z2 = jnp.sum(zt * zt, axis=0, keepdims=True)
    norm = jnp.sqrt(z2)
    mnorm = jnp.maximum(norm, 1e-12)
    znt = zt / mnorm
    x2 = jnp.sum(znt * znt, axis=0, keepdims=True)
    znt_bf = znt.astype(jnp.bfloat16)
    iota = lax.broadcasted_iota(jnp.int32, (_KC, _TM), 0)
    big = jnp.int32(2**30)

    def body(kk, carry):
        run_min, run_idx = carry
        ctb = ctm2_ref[:, pl.ds(kk * _KC, _KC)]
        c2b = c2_ref[pl.ds(kk * _KC, _KC), :]
        zct2 = lax.dot_general(ctb, znt_bf, (((0,), (0,)), ((), ())),
                               preferred_element_type=jnp.float32)
        d2 = x2 + c2b + zct2
        dd = jnp.sqrt(jnp.clip(d2, 0.0, None))
        minv = jnp.min(dd, axis=0, keepdims=True)
        idx = jnp.min(jnp.where(dd == minv, iota, big),
                      axis=0, keepdims=True) + kk * _KC
        upd = minv < run_min
        new_min = jnp.where(upd, minv, run_min)
        rounded = new_min.astype(jnp.bfloat16).astype(jnp.float32)
        new_min = jnp.where((kk % _RND) == (_RND - 1), rounded, new_min)
        return new_min, jnp.where(upd, idx, run_idx)

    init = (jnp.full((1, _TM), jnp.inf, jnp.float32),
            jnp.zeros((1, _TM), jnp.int32))
    _, run_idx = lax.fori_loop(0, _NKC, body, init)
    tok_ref[...] = run_idx.reshape(1, 1, _TM)


def _vq_argmin(zt, ctm2, c2col):
    n_rows = zt.shape[1]
    grid = n_rows // _TM
    return pl.pallas_call(
        _vq_argmin_body,
        grid=(grid,),
        in_specs=[
            pl.BlockSpec((_D, _TM), lambda i: (0, i)),
            pl.BlockSpec((_D, _CB), lambda i: (0, 0)),
            pl.BlockSpec((_CB, 1), lambda i: (0, 0)),
        ],
        out_specs=pl.BlockSpec((1, 1, _TM), lambda i: (i, 0, 0)),
        out_shape=jax.ShapeDtypeStruct((grid, 1, _TM), jnp.int32),
    )(zt, ctm2, c2col)


_DP = 128


def _make_sc_gather(n_rows):
    info = plsc.get_sparse_core_info()
    nc, ns = info.num_cores, info.num_subcores
    nw = nc * ns
    b_per_w = n_rows // nw
    mesh = plsc.VectorSubcoreMesh(core_axis_name="c", subcore_axis_name="s")

    @functools.partial(
        pl.kernel,
        out_type=jax.ShapeDtypeStruct((n_rows, _DP), jnp.float32),
        mesh=mesh,
        scratch_types=[
            pltpu.VMEM((b_per_w,), jnp.int32),
            pltpu.VMEM((b_per_w, _DP), jnp.float32),
            pltpu.SemaphoreType.DMA,
        ],
    )
    def gather_kernel(table_hbm, idx_hbm, out_hbm, idx_v, rows_v, sem):
        wid = lax.axis_index("s") * nc + lax.axis_index("c")
        base = wid * b_per_w
        pltpu.sync_copy(idx_hbm.at[pl.ds(base, b_per_w)], idx_v)
        pltpu.async_copy(table_hbm.at[idx_v], rows_v, sem).wait()
        pltpu.sync_copy(rows_v, out_hbm.at[pl.ds(base, b_per_w)])

    return gather_kernel


_LB = 1024


def _loss_body(z_ref, q_ref, loss_ref):
    diff = z_ref[...] - q_ref[...][:, :_D]
    part = jnp.sum(diff * diff)

    @pl.when(pl.program_id(0) == 0)
    def _():
        loss_ref[0, 0] = part

    @pl.when(pl.program_id(0) != 0)
    def _():
        loss_ref[0, 0] += part


def _loss_sum(z_flat, qp):
    n_rows = z_flat.shape[0]
    return pl.pallas_call(
        _loss_body,
        grid=(n_rows // _LB,),
        in_specs=[
            pl.BlockSpec((_LB, _D), lambda i: (i, 0)),
            pl.BlockSpec((_LB, _DP), lambda i: (i, 0)),
        ],
        out_specs=pl.BlockSpec(memory_space=pltpu.SMEM, block_shape=(1, 1),
                               index_map=lambda i: (0, 0)),
        out_shape=jax.ShapeDtypeStruct((1, 1), jnp.float32),
    )(z_flat, qp)


def kernel(z, codebook):
    B, N, D = z.shape
    n_rows = B * N
    z_flat = z.reshape(n_rows, D)
    zt = z_flat.T
    ct = codebook.T
    ctm2 = (-2.0 * ct).astype(jnp.bfloat16)

    c2 = _sq_norms(ct)
    tok3 = _vq_argmin(zt, ctm2, c2.reshape(_CB, 1))
    tokens_flat = tok3.reshape(-1)

    cb_pad = jnp.pad(codebook, ((0, 0), (0, _DP - D)))
    qp = _make_sc_gather(n_rows)(cb_pad, tokens_flat)
    q = qp[:, :D]

    loss_sum = _loss_sum(z_flat, qp)

    quantized = q.reshape(B, N, D)[:, None, :, :]
    tokens = tokens_flat.reshape(B, N)

    m = loss_sum[0, 0] / jnp.float32(n_rows * D)
    commitment_loss = jnp.float32(0.25) * m
    codebook_loss = m
    quantizer_loss = commitment_loss + codebook_loss
    return quantized, quantizer_loss, commitment_loss, codebook_loss, tokens


# speedup vs baseline: 2.0255x; 2.0255x over previous
import functools

import jax
import jax.numpy as jnp
from jax import lax
from jax.experimental import pallas as pl
from jax.experimental.pallas import tpu as pltpu
from jax.experimental.pallas import tpu_sc as plsc

_CB = 8192
_D = 32
_TM = 512
_KC = 2048
_NKC = _CB // _KC
_RND = 4096 // _KC


def _c2_body(ct_ref, c2_ref):
    c = ct_ref[...]
    c2_ref[...] = jnp.sum(c * c, axis=0, keepdims=True)


def _sq_norms(ct):
    return pl.pallas_call(
        _c2_body,
        grid=(1,),
        in_specs=[pl.BlockSpec((_D, _CB), lambda i: (0, 0))],
        out_specs=pl.BlockSpec((1, _CB), lambda i: (0, 0)),
        out_shape=jax.ShapeDtypeStruct((1, _CB), jnp.float32),
    )(ct)


def _vq_argmin_body(zt_ref, ctm2_ref, c2_ref, tok_ref):
    zt = zt_ref[...]
    z2 = jnp.sum(zt * zt, axis=0, keepdims=True)
    norm = jnp.sqrt(z2)
    mnorm = jnp.maximum(norm, 1e-12)
    znt = zt / mnorm
    x2 = jnp.sum(znt * znt, axis=0, keepdims=True)
    znt_bf = znt.astype(jnp.bfloat16)
    iota = lax.broadcasted_iota(jnp.int32, (_KC, _TM), 0)
    big = jnp.int32(2**30)

    run_min = jnp.full((1, _TM), jnp.inf, jnp.float32)
    run_idx = jnp.zeros((1, _TM), jnp.int32)
    for kk in range(_NKC):
        ctb = ctm2_ref[:, kk * _KC:(kk + 1) * _KC]
        c2b = c2_ref[kk * _KC:(kk + 1) * _KC, :]
        zct2 = lax.dot_general(ctb, znt_bf, (((0,), (0,)), ((), ())),
                               preferred_element_type=jnp.float32)
        d2 = x2 + c2b + zct2
        dd = jnp.sqrt(jnp.maximum(d2, 0.0))
        minv = jnp.min(dd, axis=0, keepdims=True)
        idx = jnp.min(jnp.where(dd == minv, iota, big),
                      axis=0, keepdims=True) + kk * _KC
        upd = minv < run_min
        run_idx = jnp.where(upd, idx, run_idx)
        run_min = jnp.where(upd, minv, run_min)
        if kk % _RND == _RND - 1:
            run_min = run_min.astype(jnp.bfloat16).astype(jnp.float32)
    tok_ref[...] = run_idx.reshape(1, 1, _TM)


def _vq_argmin(zt, ctm2, c2col):
    n_rows = zt.shape[1]
    grid = n_rows // _TM
    return pl.pallas_call(
        _vq_argmin_body,
        grid=(grid,),
        in_specs=[
            pl.BlockSpec((_D, _TM), lambda i: (0, i)),
            pl.BlockSpec((_D, _CB), lambda i: (0, 0)),
            pl.BlockSpec((_CB, 1), lambda i: (0, 0)),
        ],
        out_specs=pl.BlockSpec((1, 1, _TM), lambda i: (i, 0, 0)),
        out_shape=jax.ShapeDtypeStruct((grid, 1, _TM), jnp.int32),
    )(zt, ctm2, c2col)


_DP = 128


def _make_sc_gather(n_rows):
    info = plsc.get_sparse_core_info()
    nc, ns = info.num_cores, info.num_subcores
    nw = nc * ns
    b_per_w = n_rows // nw
    mesh = plsc.VectorSubcoreMesh(core_axis_name="c", subcore_axis_name="s")

    @functools.partial(
        pl.kernel,
        out_type=jax.ShapeDtypeStruct((n_rows, _DP), jnp.float32),
        mesh=mesh,
        scratch_types=[
            pltpu.VMEM((b_per_w,), jnp.int32),
            pltpu.VMEM((b_per_w, _DP), jnp.float32),
            pltpu.SemaphoreType.DMA,
        ],
    )
    def gather_kernel(table_hbm, idx_hbm, out_hbm, idx_v, rows_v, sem):
        wid = lax.axis_index("s") * nc + lax.axis_index("c")
        base = wid * b_per_w
        pltpu.sync_copy(idx_hbm.at[pl.ds(base, b_per_w)], idx_v)
        pltpu.async_copy(table_hbm.at[idx_v], rows_v, sem).wait()
        pltpu.sync_copy(rows_v, out_hbm.at[pl.ds(base, b_per_w)])

    return gather_kernel


_LB = 1024


def _loss_body(z_ref, q_ref, loss_ref):
    diff = z_ref[...] - q_ref[...][:, :_D]
    part = jnp.sum(diff * diff)

    @pl.when(pl.program_id(0) == 0)
    def _():
        loss_ref[0, 0] = part

    @pl.when(pl.program_id(0) != 0)
    def _():
        loss_ref[0, 0] += part


def _loss_sum(z_flat, qp):
    n_rows = z_flat.shape[0]
    return pl.pallas_call(
        _loss_body,
        grid=(n_rows // _LB,),
        in_specs=[
            pl.BlockSpec((_LB, _D), lambda i: (i, 0)),
            pl.BlockSpec((_LB, _DP), lambda i: (i, 0)),
        ],
        out_specs=pl.BlockSpec(memory_space=pltpu.SMEM, block_shape=(1, 1),
                               index_map=lambda i: (0, 0)),
        out_shape=jax.ShapeDtypeStruct((1, 1), jnp.float32),
    )(z_flat, qp)


def kernel(z, codebook):
    B, N, D = z.shape
    n_rows = B * N
    z_flat = z.reshape(n_rows, D)
    zt = z_flat.T
    ct = codebook.T
    ctm2 = (-2.0 * ct).astype(jnp.bfloat16)

    c2 = _sq_norms(ct)
    tok3 = _vq_argmin(zt, ctm2, c2.reshape(_CB, 1))
    tokens_flat = tok3.reshape(-1)

    cb_pad = jnp.pad(codebook, ((0, 0), (0, _DP - D)))
    qp = _make_sc_gather(n_rows)(cb_pad, tokens_flat)
    q = qp[:, :D]

    loss_sum = _loss_sum(z_flat, qp)

    quantized = q.reshape(B, N, D)[:, None, :, :]
    tokens = tokens_flat.reshape(B, N)

    m = loss_sum[0, 0] / jnp.float32(n_rows * D)
    commitment_loss = jnp.float32(0.25) * m
    codebook_loss = m
    quantizer_loss = commitment_loss + codebook_loss
    return quantized, quantizer_loss, commitment_loss, codebook_loss, tokens


# speedup vs baseline: 2.4693x; 1.2191x over previous
import functools

import jax
import jax.numpy as jnp
from jax import lax
from jax.experimental import pallas as pl
from jax.experimental.pallas import tpu as pltpu
from jax.experimental.pallas import tpu_sc as plsc

_CB = 8192
_D = 32
_TM = 512
_KC = 2048
_NKC = _CB // _KC
_RND = 4096 // _KC


def _c2_body(ct_ref, c2_ref):
    c = ct_ref[...]
    c2_ref[...] = jnp.sum(c * c, axis=0, keepdims=True)


def _sq_norms(ct):
    return pl.pallas_call(
        _c2_body,
        grid=(1,),
        in_specs=[pl.BlockSpec((_D, _CB), lambda i: (0, 0))],
        out_specs=pl.BlockSpec((1, _CB), lambda i: (0, 0)),
        out_shape=jax.ShapeDtypeStruct((1, _CB), jnp.float32),
    )(ct)


def _vq_argmin_body(zt_ref, ctm2_ref, c2_ref, tok_ref):
    zt = zt_ref[...]
    z2 = jnp.sum(zt * zt, axis=0, keepdims=True)
    norm = jnp.sqrt(z2)
    mnorm = jnp.maximum(norm, 1e-12)
    znt = zt / mnorm
    x2 = jnp.sum(znt * znt, axis=0, keepdims=True)
    znt_bf = znt.astype(jnp.bfloat16)
    iota = lax.broadcasted_iota(jnp.int32, (_KC, _TM), 0)
    big = jnp.int32(2**30)

    run_min = jnp.full((1, _TM), jnp.inf, jnp.float32)
    run_idx = jnp.zeros((1, _TM), jnp.int32)
    for kk in range(_NKC):
        ctb = ctm2_ref[:, kk * _KC:(kk + 1) * _KC]
        c2b = c2_ref[kk * _KC:(kk + 1) * _KC, :]
        zct2 = lax.dot_general(ctb, znt_bf, (((0,), (0,)), ((), ())),
                               preferred_element_type=jnp.float32)
        d2 = x2 + c2b + zct2
        dd = jnp.where(d2 <= 0.0, 0.0, d2 * lax.rsqrt(d2))
        minv = jnp.min(dd, axis=0, keepdims=True)
        idx = jnp.min(jnp.where(dd == minv, iota, big),
                      axis=0, keepdims=True) + kk * _KC
        upd = minv < run_min
        run_idx = jnp.where(upd, idx, run_idx)
        run_min = jnp.where(upd, minv, run_min)
        if kk % _RND == _RND - 1:
            run_min = run_min.astype(jnp.bfloat16).astype(jnp.float32)
    tok_ref[...] = run_idx.reshape(1, 1, _TM)


def _vq_argmin(zt, ctm2, c2col):
    n_rows = zt.shape[1]
    grid = n_rows // _TM
    return pl.pallas_call(
        _vq_argmin_body,
        grid=(grid,),
        in_specs=[
            pl.BlockSpec((_D, _TM), lambda i: (0, i)),
            pl.BlockSpec((_D, _CB), lambda i: (0, 0)),
            pl.BlockSpec((_CB, 1), lambda i: (0, 0)),
        ],
        out_specs=pl.BlockSpec((1, 1, _TM), lambda i: (i, 0, 0)),
        out_shape=jax.ShapeDtypeStruct((grid, 1, _TM), jnp.int32),
    )(zt, ctm2, c2col)


_DP = 128


def _make_sc_gather(n_rows):
    info = plsc.get_sparse_core_info()
    nc, ns = info.num_cores, info.num_subcores
    nw = nc * ns
    b_per_w = n_rows // nw
    mesh = plsc.VectorSubcoreMesh(core_axis_name="c", subcore_axis_name="s")

    @functools.partial(
        pl.kernel,
        out_type=jax.ShapeDtypeStruct((n_rows, _DP), jnp.float32),
        mesh=mesh,
        scratch_types=[
            pltpu.VMEM((b_per_w,), jnp.int32),
            pltpu.VMEM((b_per_w, _DP), jnp.float32),
            pltpu.SemaphoreType.DMA,
        ],
    )
    def gather_kernel(table_hbm, idx_hbm, out_hbm, idx_v, rows_v, sem):
        wid = lax.axis_index("s") * nc + lax.axis_index("c")
        base = wid * b_per_w
        pltpu.sync_copy(idx_hbm.at[pl.ds(base, b_per_w)], idx_v)
        pltpu.async_copy(table_hbm.at[idx_v], rows_v, sem).wait()
        pltpu.sync_copy(rows_v, out_hbm.at[pl.ds(base, b_per_w)])

    return gather_kernel


_LB = 1024


def _loss_body(z_ref, q_ref, loss_ref):
    diff = z_ref[...] - q_ref[...][:, :_D]
    part = jnp.sum(diff * diff)

    @pl.when(pl.program_id(0) == 0)
    def _():
        loss_ref[0, 0] = part

    @pl.when(pl.program_id(0) != 0)
    def _():
        loss_ref[0, 0] += part


def _loss_sum(z_flat, qp):
    n_rows = z_flat.shape[0]
    return pl.pallas_call(
        _loss_body,
        grid=(n_rows // _LB,),
        in_specs=[
            pl.BlockSpec((_LB, _D), lambda i: (i, 0)),
            pl.BlockSpec((_LB, _DP), lambda i: (i, 0)),
        ],
        out_specs=pl.BlockSpec(memory_space=pltpu.SMEM, block_shape=(1, 1),
                               index_map=lambda i: (0, 0)),
        out_shape=jax.ShapeDtypeStruct((1, 1), jnp.float32),
    )(z_flat, qp)


def kernel(z, codebook):
    B, N, D = z.shape
    n_rows = B * N
    z_flat = z.reshape(n_rows, D)
    zt = z_flat.T
    ct = codebook.T
    ctm2 = (-2.0 * ct).astype(jnp.bfloat16)

    c2 = _sq_norms(ct)
    tok3 = _vq_argmin(zt, ctm2, c2.reshape(_CB, 1))
    tokens_flat = tok3.reshape(-1)

    cb_pad = jnp.pad(codebook, ((0, 0), (0, _DP - D)))
    qp = _make_sc_gather(n_rows)(cb_pad, tokens_flat)
    q = qp[:, :D]

    loss_sum = _loss_sum(z_flat, qp)

    quantized = q.reshape(B, N, D)[:, None, :, :]
    tokens = tokens_flat.reshape(B, N)

    m = loss_sum[0, 0] / jnp.float32(n_rows * D)
    commitment_loss = jnp.float32(0.25) * m
    codebook_loss = m
    quantizer_loss = commitment_loss + codebook_loss
    return quantized, quantizer_loss, commitment_loss, codebook_loss, tokens


# speedup vs baseline: 2.5982x; 1.0522x over previous
import functools

import jax
import jax.numpy as jnp
from jax import lax
from jax.experimental import pallas as pl
from jax.experimental.pallas import tpu as pltpu
from jax.experimental.pallas import tpu_sc as plsc

_CB = 8192
_D = 32
_TM = 1024
_KC = 2048
_NKC = _CB // _KC
_RND = 4096 // _KC


def _c2_body(ct_ref, c2_ref):
    c = ct_ref[...]
    c2_ref[...] = jnp.sum(c * c, axis=0, keepdims=True)


def _sq_norms(ct):
    return pl.pallas_call(
        _c2_body,
        grid=(1,),
        in_specs=[pl.BlockSpec((_D, _CB), lambda i: (0, 0))],
        out_specs=pl.BlockSpec((1, _CB), lambda i: (0, 0)),
        out_shape=jax.ShapeDtypeStruct((1, _CB), jnp.float32),
    )(ct)


def _vq_argmin_body(zt_ref, ctm2_ref, c2_ref, tok_ref):
    zt = zt_ref[...]
    z2 = jnp.sum(zt * zt, axis=0, keepdims=True)
    norm = jnp.sqrt(z2)
    mnorm = jnp.maximum(norm, 1e-12)
    znt = zt / mnorm
    x2 = jnp.sum(znt * znt, axis=0, keepdims=True)
    znt_bf = znt.astype(jnp.bfloat16)
    iota = lax.broadcasted_iota(jnp.int32, (_KC, _TM), 0)
    big = jnp.int32(2**30)

    run_min = jnp.full((1, _TM), jnp.inf, jnp.float32)
    run_idx = jnp.zeros((1, _TM), jnp.int32)
    for kk in range(_NKC):
        ctb = ctm2_ref[:, kk * _KC:(kk + 1) * _KC]
        c2b = c2_ref[kk * _KC:(kk + 1) * _KC, :]
        zct2 = lax.dot_general(ctb, znt_bf, (((0,), (0,)), ((), ())),
                               preferred_element_type=jnp.float32)
        d2 = x2 + c2b + zct2
        dd = jnp.where(d2 <= 0.0, 0.0, d2 * lax.rsqrt(d2))
        minv = jnp.min(dd, axis=0, keepdims=True)
        idx = jnp.min(jnp.where(dd == minv, iota, big),
                      axis=0, keepdims=True) + kk * _KC
        upd = minv < run_min
        run_idx = jnp.where(upd, idx, run_idx)
        run_min = jnp.where(upd, minv, run_min)
        if kk % _RND == _RND - 1:
            run_min = run_min.astype(jnp.bfloat16).astype(jnp.float32)
    tok_ref[...] = run_idx.reshape(1, 1, _TM)


def _vq_argmin(zt, ctm2, c2col):
    n_rows = zt.shape[1]
    grid = n_rows // _TM
    return pl.pallas_call(
        _vq_argmin_body,
        grid=(grid,),
        in_specs=[
            pl.BlockSpec((_D, _TM), lambda i: (0, i)),
            pl.BlockSpec((_D, _CB), lambda i: (0, 0)),
            pl.BlockSpec((_CB, 1), lambda i: (0, 0)),
        ],
        out_specs=pl.BlockSpec((1, 1, _TM), lambda i: (i, 0, 0)),
        out_shape=jax.ShapeDtypeStruct((grid, 1, _TM), jnp.int32),
    )(zt, ctm2, c2col)


_DP = 128


def _make_sc_gather(n_rows):
    info = plsc.get_sparse_core_info()
    nc, ns = info.num_cores, info.num_subcores
    nw = nc * ns
    b_per_w = n_rows // nw
    mesh = plsc.VectorSubcoreMesh(core_axis_name="c", subcore_axis_name="s")

    @functools.partial(
        pl.kernel,
        out_type=jax.ShapeDtypeStruct((n_rows, _DP), jnp.float32),
        mesh=mesh,
        scratch_types=[
            pltpu.VMEM((b_per_w,), jnp.int32),
            pltpu.VMEM((b_per_w, _DP), jnp.float32),
            pltpu.SemaphoreType.DMA,
        ],
    )
    def gather_kernel(table_hbm, idx_hbm, out_hbm, idx_v, rows_v, sem):
        wid = lax.axis_index("s") * nc + lax.axis_index("c")
        base = wid * b_per_w
        pltpu.sync_copy(idx_hbm.at[pl.ds(base, b_per_w)], idx_v)
        pltpu.async_copy(table_hbm.at[idx_v], rows_v, sem).wait()
        pltpu.sync_copy(rows_v, out_hbm.at[pl.ds(base, b_per_w)])

    return gather_kernel


_LB = 1024


def _loss_body(z_ref, q_ref, loss_ref):
    diff = z_ref[...] - q_ref[...][:, :_D]
    part = jnp.sum(diff * diff)

    @pl.when(pl.program_id(0) == 0)
    def _():
        loss_ref[0, 0] = part

    @pl.when(pl.program_id(0) != 0)
    def _():
        loss_ref[0, 0] += part


def _loss_sum(z_flat, qp):
    n_rows = z_flat.shape[0]
    return pl.pallas_call(
        _loss_body,
        grid=(n_rows // _LB,),
        in_specs=[
            pl.BlockSpec((_LB, _D), lambda i: (i, 0)),
            pl.BlockSpec((_LB, _DP), lambda i: (i, 0)),
        ],
        out_specs=pl.BlockSpec(memory_space=pltpu.SMEM, block_shape=(1, 1),
                               index_map=lambda i: (0, 0)),
        out_shape=jax.ShapeDtypeStruct((1, 1), jnp.float32),
    )(z_flat, qp)


def kernel(z, codebook):
    B, N, D = z.shape
    n_rows = B * N
    z_flat = z.reshape(n_rows, D)
    zt = z_flat.T
    ct = codebook.T
    ctm2 = (-2.0 * ct).astype(jnp.bfloat16)

    c2 = _sq_norms(ct)
    tok3 = _vq_argmin(zt, ctm2, c2.reshape(_CB, 1))
    tokens_flat = tok3.reshape(-1)

    cb_pad = jnp.pad(codebook, ((0, 0), (0, _DP - D)))
    qp = _make_sc_gather(n_rows)(cb_pad, tokens_flat)
    q = qp[:, :D]

    loss_sum = _loss_sum(z_flat, qp)

    quantized = q.reshape(B, N, D)[:, None, :, :]
    tokens = tokens_flat.reshape(B, N)

    m = loss_sum[0, 0] / jnp.float32(n_rows * D)
    commitment_loss = jnp.float32(0.25) * m
    codebook_loss = m
    quantizer_loss = commitment_loss + codebook_loss
    return quantized, quantizer_loss, commitment_loss, codebook_loss, tokens
